# Initial kernel scaffold; baseline (speedup 1.0000x reference)
import jax
import jax.numpy as jnp
from jax.experimental import pallas as pl


def kernel(word_inputs, char_inputs, char_seq_lengths, char_seq_recover, word_embedding):
    raise NotImplementedError("write your pallas kernel here")


# speedup vs baseline: 5.7790x; 5.7790x over previous
import functools

import jax
import jax.numpy as jnp
from jax import lax
from jax.experimental import pallas as pl
from jax.experimental.pallas import tpu as pltpu
from jax.experimental.pallas import tpu_sc as plsc

VOCAB = 100000
EMBED_DIM = 128
BATCH = 1024
SENT_LEN = 200

N_ROWS = BATCH * SENT_LEN
NUM_CORES = 2
NUM_SUBCORES = 16
NW = NUM_CORES * NUM_SUBCORES
ROWS_PER_W = N_ROWS // NW
CHUNK = 128
N_CHUNKS = ROWS_PER_W // CHUNK

_mesh = plsc.VectorSubcoreMesh(core_axis_name="c", subcore_axis_name="s")


@functools.partial(
    pl.kernel,
    mesh=_mesh,
    out_type=jax.ShapeDtypeStruct((NW, N_CHUNKS, CHUNK, EMBED_DIM), jnp.float32),
    scratch_types=[
        pltpu.VMEM((N_CHUNKS, CHUNK), jnp.int32),
        pltpu.VMEM((CHUNK, EMBED_DIM), jnp.float32),
        pltpu.SemaphoreType.DMA,
    ],
)
def _embedding_gather(table_hbm, idx_hbm, out_hbm, idx_v, rows_v, sem):
    wid = lax.axis_index("s") * NUM_CORES + lax.axis_index("c")
    pltpu.sync_copy(idx_hbm.at[wid], idx_v)

    def body(j, carry):
        pltpu.async_copy(table_hbm.at[idx_v.at[j]], rows_v, sem).wait()
        pltpu.sync_copy(rows_v, out_hbm.at[wid, j])
        return carry

    lax.fori_loop(0, N_CHUNKS, body, 0)


def kernel(word_inputs, char_inputs, char_seq_lengths, char_seq_recover, word_embedding):
    idx = word_inputs.reshape(NW, N_CHUNKS, CHUNK).astype(jnp.int32)
    out = _embedding_gather(word_embedding, idx)
    return out.reshape(BATCH, SENT_LEN, EMBED_DIM)


# speedup vs baseline: 7.7927x; 1.3484x over previous
import functools

import jax
import jax.numpy as jnp
from jax import lax
from jax.experimental import pallas as pl
from jax.experimental.pallas import tpu as pltpu
from jax.experimental.pallas import tpu_sc as plsc

VOCAB = 100000
EMBED_DIM = 128
BATCH = 1024
SENT_LEN = 200

N_ROWS = BATCH * SENT_LEN
NUM_CORES = 2
NUM_SUBCORES = 16
NW = NUM_CORES * NUM_SUBCORES
ROWS_PER_W = N_ROWS // NW
CHUNK = 128
N_CHUNKS = ROWS_PER_W // CHUNK
NBUF = 5
N_GROUPS = N_CHUNKS // NBUF

_mesh = plsc.VectorSubcoreMesh(core_axis_name="c", subcore_axis_name="s")


@functools.partial(
    pl.kernel,
    mesh=_mesh,
    out_type=jax.ShapeDtypeStruct((NW, N_CHUNKS, CHUNK, EMBED_DIM), jnp.float32),
    scratch_types=[pltpu.VMEM((N_CHUNKS, CHUNK), jnp.int32)]
    + [pltpu.VMEM((CHUNK, EMBED_DIM), jnp.float32) for _ in range(NBUF)]
    + [pltpu.SemaphoreType.DMA for _ in range(2 * NBUF)],
)
def _embedding_gather(table_hbm, idx_hbm, out_hbm, idx_v,
                      b0, b1, b2, b3, b4,
                      g0, g1, g2, g3, g4,
                      o0, o1, o2, o3, o4):
    bufs = (b0, b1, b2, b3, b4)
    gsems = (g0, g1, g2, g3, g4)
    osems = (o0, o1, o2, o3, o4)
    wid = lax.axis_index("s") * NUM_CORES + lax.axis_index("c")
    pltpu.sync_copy(idx_hbm.at[wid], idx_v)

    for b in range(NBUF):
        pltpu.async_copy(table_hbm.at[idx_v.at[b]], bufs[b], gsems[b])

    def body(t, carry):
        base = t * NBUF
        for b in range(NBUF):
            j = base + b
            pltpu.make_async_copy(table_hbm.at[idx_v.at[j]], bufs[b], gsems[b]).wait()
            pltpu.async_copy(bufs[b], out_hbm.at[wid, j], osems[b])
        for b in range(NBUF):
            jn = base + b + NBUF

            @pl.when(jn < N_CHUNKS)
            def _():
                pltpu.make_async_copy(bufs[b], out_hbm.at[wid, 0], osems[b]).wait()
                pltpu.async_copy(table_hbm.at[idx_v.at[jn]], bufs[b], gsems[b])

        return carry

    lax.fori_loop(0, N_GROUPS, body, 0)
    for b in range(NBUF):
        pltpu.make_async_copy(bufs[b], out_hbm.at[wid, 0], osems[b]).wait()


def kernel(word_inputs, char_inputs, char_seq_lengths, char_seq_recover, word_embedding):
    idx = word_inputs.reshape(NW, N_CHUNKS, CHUNK).astype(jnp.int32)
    out = _embedding_gather(word_embedding, idx)
    return out.reshape(BATCH, SENT_LEN, EMBED_DIM)


# speedup vs baseline: 7.8676x; 1.0096x over previous
import functools

import jax
import jax.numpy as jnp
from jax import lax
from jax.experimental import pallas as pl
from jax.experimental.pallas import tpu as pltpu
from jax.experimental.pallas import tpu_sc as plsc

VOCAB = 100000
EMBED_DIM = 128
BATCH = 1024
SENT_LEN = 200

N_ROWS = BATCH * SENT_LEN
NUM_CORES = 2
NUM_SUBCORES = 16
NW = NUM_CORES * NUM_SUBCORES
ROWS_PER_W = N_ROWS // NW
CHUNK = 128
N_CHUNKS = ROWS_PER_W // CHUNK
NBUF = 7
N_GROUPS = -(-N_CHUNKS // NBUF)

_mesh = plsc.VectorSubcoreMesh(core_axis_name="c", subcore_axis_name="s")


@functools.partial(
    pl.kernel,
    mesh=_mesh,
    out_type=jax.ShapeDtypeStruct((NW, N_CHUNKS, CHUNK, EMBED_DIM), jnp.float32),
    scratch_types=[pltpu.VMEM((N_CHUNKS, CHUNK), jnp.int32)]
    + [pltpu.VMEM((CHUNK, EMBED_DIM), jnp.float32) for _ in range(NBUF)]
    + [pltpu.SemaphoreType.DMA for _ in range(2 * NBUF)],
)
def _embedding_gather(table_hbm, idx_hbm, out_hbm, idx_v,
                      b0, b1, b2, b3, b4, b5, b6,
                      g0, g1, g2, g3, g4, g5, g6,
                      o0, o1, o2, o3, o4, o5, o6):
    bufs = (b0, b1, b2, b3, b4, b5, b6)
    gsems = (g0, g1, g2, g3, g4, g5, g6)
    osems = (o0, o1, o2, o3, o4, o5, o6)
    wid = lax.axis_index("s") * NUM_CORES + lax.axis_index("c")
    pltpu.sync_copy(idx_hbm.at[wid], idx_v)

    for b in range(NBUF):
        pltpu.async_copy(table_hbm.at[idx_v.at[b]], bufs[b], gsems[b])

    def body(t, carry):
        base = t * NBUF
        for b in range(NBUF):
            j = base + b

            @pl.when(j < N_CHUNKS)
            def _():
                pltpu.make_async_copy(table_hbm.at[idx_v.at[j]], bufs[b], gsems[b]).wait()
                pltpu.async_copy(bufs[b], out_hbm.at[wid, j], osems[b])

        for b in range(NBUF):
            jn = base + b + NBUF

            @pl.when(jn < N_CHUNKS)
            def _():
                pltpu.make_async_copy(bufs[b], out_hbm.at[wid, 0], osems[b]).wait()
                pltpu.async_copy(table_hbm.at[idx_v.at[jn]], bufs[b], gsems[b])

        return carry

    lax.fori_loop(0, N_GROUPS, body, 0)
    for b in range(NBUF):
        pltpu.make_async_copy(bufs[b], out_hbm.at[wid, 0], osems[b]).wait()


def kernel(word_inputs, char_inputs, char_seq_lengths, char_seq_recover, word_embedding):
    idx = word_inputs.reshape(NW, N_CHUNKS, CHUNK).astype(jnp.int32)
    out = _embedding_gather(word_embedding, idx)
    return out.reshape(BATCH, SENT_LEN, EMBED_DIM)


# speedup vs baseline: 7.8863x; 1.0024x over previous
import functools

import jax
import jax.numpy as jnp
from jax import lax
from jax.experimental import pallas as pl
from jax.experimental.pallas import tpu as pltpu
from jax.experimental.pallas import tpu_sc as plsc

VOCAB = 100000
EMBED_DIM = 128
BATCH = 1024
SENT_LEN = 200

N_ROWS = BATCH * SENT_LEN
NUM_CORES = 2
NUM_SUBCORES = 16
NW = NUM_CORES * NUM_SUBCORES
ROWS_PER_W = N_ROWS // NW
CHUNK = 64
N_CHUNKS = ROWS_PER_W // CHUNK
NBUF = 14
N_GROUPS = -(-N_CHUNKS // NBUF)

_mesh = plsc.VectorSubcoreMesh(core_axis_name="c", subcore_axis_name="s")


@functools.partial(
    pl.kernel,
    mesh=_mesh,
    out_type=jax.ShapeDtypeStruct((NW, N_CHUNKS, CHUNK, EMBED_DIM), jnp.float32),
    scratch_types=[pltpu.VMEM((N_CHUNKS, CHUNK), jnp.int32)]
    + [pltpu.VMEM((CHUNK, EMBED_DIM), jnp.float32) for _ in range(NBUF)]
    + [pltpu.SemaphoreType.DMA for _ in range(2 * NBUF)],
)
def _embedding_gather(table_hbm, idx_hbm, out_hbm, idx_v, *rest):
    bufs = rest[:NBUF]
    gsems = rest[NBUF:2 * NBUF]
    osems = rest[2 * NBUF:]
    wid = lax.axis_index("s") * NUM_CORES + lax.axis_index("c")
    pltpu.sync_copy(idx_hbm.at[wid], idx_v)

    for b in range(NBUF):
        pltpu.async_copy(table_hbm.at[idx_v.at[b]], bufs[b], gsems[b])

    def body(t, carry):
        base = t * NBUF
        for b in range(NBUF):
            j = base + b

            @pl.when(j < N_CHUNKS)
            def _():
                pltpu.make_async_copy(table_hbm.at[idx_v.at[j]], bufs[b], gsems[b]).wait()
                pltpu.async_copy(bufs[b], out_hbm.at[wid, j], osems[b])

        for b in range(NBUF):
            jn = base + b + NBUF

            @pl.when(jn < N_CHUNKS)
            def _():
                pltpu.make_async_copy(bufs[b], out_hbm.at[wid, 0], osems[b]).wait()
                pltpu.async_copy(table_hbm.at[idx_v.at[jn]], bufs[b], gsems[b])

        return carry

    lax.fori_loop(0, N_GROUPS, body, 0)
    for b in range(NBUF):
        pltpu.make_async_copy(bufs[b], out_hbm.at[wid, 0], osems[b]).wait()


def kernel(word_inputs, char_inputs, char_seq_lengths, char_seq_recover, word_embedding):
    idx = word_inputs.reshape(NW, N_CHUNKS, CHUNK).astype(jnp.int32)
    out = _embedding_gather(word_embedding, idx)
    return out.reshape(BATCH, SENT_LEN, EMBED_DIM)
